# baseline (speedup 1.0000x reference)
import functools

import jax
import jax.numpy as jnp
from jax import lax
from jax.experimental import pallas as pl
from jax.experimental.pallas import tpu as pltpu
from jax.experimental.pallas import tpu_sc as plsc

NC = 2
NS = 16
NW = NC * NS
LANES = 16


@functools.lru_cache(maxsize=None)
def _make_sc_lookup(N, D, n_chunks, gath):
    per_w = N // NW
    chunk = per_w // n_chunks
    assert per_w * NW == N and chunk * n_chunks == per_w
    assert chunk % gath == 0 and chunk % LANES == 0 and chunk % 8 == 0

    mesh = plsc.VectorSubcoreMesh(
        core_axis_name="c", subcore_axis_name="s", num_cores=NC, num_subcores=NS
    )

    @functools.partial(
        pl.kernel,
        out_type=jax.ShapeDtypeStruct((N, D), jnp.float32),
        mesh=mesh,
        scratch_types=[
            pltpu.VMEM((chunk,), jnp.int32),
            pltpu.VMEM((chunk,), jnp.int32),
            pltpu.VMEM((chunk,), jnp.int32),
            pltpu.VMEM((chunk, D), jnp.float32),
            pltpu.SemaphoreType.DMA,
        ],
        compiler_params=pltpu.CompilerParams(use_tc_tiling_on_sc=False),
    )
    def body(xcat_hbm, offs_hbm, table_hbm, out_hbm, xc_v, of_v, idx_v, rows_v, sem):
        wid = lax.axis_index("s") * NC + lax.axis_index("c")
        base = wid * per_w

        def do_chunk(k, carry):
            cbase = base + k * chunk
            pltpu.sync_copy(xcat_hbm.at[pl.ds(cbase, chunk)], xc_v)
            pltpu.sync_copy(offs_hbm.at[pl.ds(cbase, chunk)], of_v)

            def add_vec(i, c):
                sl = pl.ds(i * LANES, LANES)
                idx_v[sl] = xc_v[sl] + of_v[sl]
                return c

            lax.fori_loop(0, chunk // LANES, add_vec, 0)

            def do_gather(j, c):
                sl = pl.ds(j * gath, gath)
                pltpu.async_copy(
                    table_hbm.at[idx_v.at[sl]], rows_v.at[sl], sem
                ).wait()
                return c

            lax.fori_loop(0, chunk // gath, do_gather, 0)

            pltpu.sync_copy(rows_v, out_hbm.at[pl.ds(cbase, chunk)])
            return carry

        lax.fori_loop(0, n_chunks, do_chunk, 0)

    return body


def kernel(xCat, table, offsets):
    B, F = xCat.shape
    _, D = table.shape
    N = B * F
    x_flat = xCat.reshape(N)
    offs_flat = jnp.tile(offsets.astype(jnp.int32), B)
    out = _make_sc_lookup(N, D, 8, 128)(x_flat, offs_flat, table)
    return out.reshape(B, F, D)


# speedup vs baseline: 1.0353x; 1.0353x over previous
import functools

import jax
import jax.numpy as jnp
from jax import lax
from jax.experimental import pallas as pl
from jax.experimental.pallas import tpu as pltpu
from jax.experimental.pallas import tpu_sc as plsc

NC = 2
NS = 16
NW = NC * NS
LANES = 16


@functools.lru_cache(maxsize=None)
def _make_sc_lookup(N, D, n_chunks, gath):
    per_w = N // NW
    chunk = per_w // n_chunks
    assert per_w * NW == N and chunk * n_chunks == per_w
    assert chunk % gath == 0 and chunk % LANES == 0 and chunk % 8 == 0

    mesh = plsc.VectorSubcoreMesh(
        core_axis_name="c", subcore_axis_name="s", num_cores=NC, num_subcores=NS
    )

    @functools.partial(
        pl.kernel,
        out_type=jax.ShapeDtypeStruct((N, D), jnp.float32),
        mesh=mesh,
        scratch_types=[
            pltpu.VMEM((chunk,), jnp.int32),
            pltpu.VMEM((chunk,), jnp.int32),
            pltpu.VMEM((chunk,), jnp.int32),
            pltpu.VMEM((chunk, D), jnp.float32),
            pltpu.SemaphoreType.DMA,
        ],
        compiler_params=pltpu.CompilerParams(use_tc_tiling_on_sc=False),
    )
    def body(xcat_hbm, offs_hbm, table_hbm, out_hbm, xc_v, of_v, idx_v, rows_v, sem):
        wid = lax.axis_index("s") * NC + lax.axis_index("c")
        base = wid * per_w

        def do_chunk(k, carry):
            cbase = base + k * chunk
            pltpu.sync_copy(xcat_hbm.at[pl.ds(cbase, chunk)], xc_v)
            pltpu.sync_copy(offs_hbm.at[pl.ds(cbase, chunk)], of_v)

            def add_vec(i, c):
                sl = pl.ds(i * LANES, LANES)
                idx_v[sl] = xc_v[sl] + of_v[sl]
                return c

            lax.fori_loop(0, chunk // LANES, add_vec, 0)

            def do_gather(j, c):
                sl = pl.ds(j * gath, gath)
                pltpu.async_copy(
                    table_hbm.at[idx_v.at[sl]], rows_v.at[sl], sem
                ).wait()
                return c

            lax.fori_loop(0, chunk // gath, do_gather, 0)

            pltpu.sync_copy(rows_v, out_hbm.at[pl.ds(cbase, chunk)])
            return carry

        lax.fori_loop(0, n_chunks, do_chunk, 0)

    return body


def kernel(xCat, table, offsets):
    B, F = xCat.shape
    _, D = table.shape
    N = B * F
    x_flat = xCat.reshape(N)
    offs_flat = jnp.tile(offsets.astype(jnp.int32), B)
    out = _make_sc_lookup(N, D, 8, 1664)(x_flat, offs_flat, table)
    return out.reshape(B, F, D)


# speedup vs baseline: 1.3224x; 1.2773x over previous
import functools

import jax
import jax.numpy as jnp
from jax import lax
from jax.experimental import pallas as pl
from jax.experimental.pallas import tpu as pltpu
from jax.experimental.pallas import tpu_sc as plsc

NC = 2
NS = 16
NW = NC * NS
LANES = 16


@functools.lru_cache(maxsize=None)
def _make_relayout(V, D, B, F):
    n_cols_full = V // 128
    tail = V - n_cols_full * 128
    N = B * F
    per_w = N // NW
    base_cols = n_cols_full // NW
    extra = n_cols_full - base_cols * NW

    mesh = plsc.VectorSubcoreMesh(
        core_axis_name="c", subcore_axis_name="s", num_cores=NC, num_subcores=NS
    )

    @functools.partial(
        pl.kernel,
        out_type=(
            jax.ShapeDtypeStruct((V * D // 128, 128), jnp.float32),
            jax.ShapeDtypeStruct((N,), jnp.int32),
        ),
        mesh=mesh,
        scratch_types=[
            pltpu.VMEM((2, D, 128), jnp.float32),
            pltpu.VMEM((2, 16, 128), jnp.float32),
            pltpu.VMEM((per_w,), jnp.int32),
            pltpu.VMEM((32,), jnp.int32),
            pltpu.SemaphoreType.DMA,
            pltpu.SemaphoreType.DMA,
        ],
        compiler_params=pltpu.CompilerParams(
            use_tc_tiling_on_sc=True, needs_layout_passes=False
        ),
    )
    def body(tt_hbm, xt_hbm, offs_hbm, scratch_hbm, rflat_hbm,
             tin, tout, xbuf, obuf, in_sem, out_sem):
        wid = lax.axis_index("s") * NC + lax.axis_index("c")
        my_cols = jnp.where(wid < extra, base_cols + 1, base_cols)
        col0 = wid * base_cols + jnp.minimum(wid, extra)

        def start_in(i):
            c = col0 + i
            return pltpu.async_copy(
                tt_hbm.at[:, pl.ds(c * 128, 128)], tin.at[i % 2], in_sem
            )

        def transpose_block(src, dst, nrows):
            def rows8(m, carry):
                for u in range(8):
                    r = m * 8 + u
                    idx_l = jnp.full((LANES,), r, jnp.int32)
                    vals = plsc.load_gather(
                        src, [lax.iota(jnp.int32, LANES), idx_l]
                    )
                    dst[m, pl.ds(u * 16, 16)] = vals
                return carry

            lax.fori_loop(0, nrows // 8, rows8, 0)

        start_in(0)

        def do_col(i, carry):
            c = col0 + i
            pltpu.make_async_copy(
                tt_hbm.at[:, pl.ds(c * 128, 128)], tin.at[i % 2], in_sem
            ).wait()

            @pl.when(i + 1 < my_cols)
            def _():
                pltpu.async_copy(
                    tt_hbm.at[:, pl.ds((c + 1) * 128, 128)], tin.at[(i + 1) % 2],
                    in_sem,
                )

            @pl.when(i >= 1)
            def _():
                pltpu.make_async_copy(
                    tout.at[(i - 1) % 2],
                    scratch_hbm.at[pl.ds((c - 1) * 16, 16)],
                    out_sem,
                ).wait()

            transpose_block(tin.at[i % 2], tout.at[i % 2], 128)
            pltpu.async_copy(
                tout.at[i % 2], scratch_hbm.at[pl.ds(c * 16, 16)], out_sem
            )
            return carry

        lax.fori_loop(0, my_cols, do_col, 0)
        pltpu.make_async_copy(
            tout.at[(my_cols - 1) % 2],
            scratch_hbm.at[pl.ds((col0 + my_cols - 1) * 16, 16)],
            out_sem,
        ).wait()

        pltpu.sync_copy(offs_hbm, obuf.at[pl.ds(0, F)])
        p0 = wid * per_w
        SUB = 1024
        assert per_w % SUB == 0 and B % SUB == 0

        def sub(s, carry):
            p = p0 + s * SUB
            f = p // B
            b0 = p - f * B
            pltpu.sync_copy(
                xt_hbm.at[f, pl.ds(b0, SUB)],
                xbuf.at[pl.ds(s * SUB, SUB)],
            )
            offv = plsc.load_gather(obuf, [jnp.full((LANES,), f, jnp.int32)])

            def addv(i, c):
                sl = pl.ds(s * SUB + i * LANES, LANES)
                xbuf[sl] = xbuf[sl] + offv
                return c

            lax.fori_loop(0, SUB // LANES, addv, 0)
            return carry

        lax.fori_loop(0, per_w // SUB, sub, 0)
        pltpu.sync_copy(xbuf, rflat_hbm.at[pl.ds(p0, per_w)])

    return body


@functools.lru_cache(maxsize=None)
def _make_gather(N, D, n_chunks, gath):
    per_w = N // NW
    chunk = per_w // n_chunks
    assert per_w * NW == N and chunk * n_chunks == per_w
    assert chunk % gath == 0 and chunk % 8 == 0

    mesh = plsc.VectorSubcoreMesh(
        core_axis_name="c", subcore_axis_name="s", num_cores=NC, num_subcores=NS
    )

    @functools.partial(
        pl.kernel,
        out_type=jax.ShapeDtypeStruct((N, D), jnp.float32),
        mesh=mesh,
        scratch_types=[
            pltpu.VMEM((chunk,), jnp.int32),
            pltpu.VMEM((chunk,), jnp.int32),
            pltpu.VMEM((chunk, D), jnp.float32),
            pltpu.VMEM((chunk, D), jnp.float32),
            pltpu.SemaphoreType.DMA,
            pltpu.SemaphoreType.DMA,
            pltpu.SemaphoreType.DMA,
            pltpu.SemaphoreType.DMA,
            pltpu.SemaphoreType.DMA,
        ],
        compiler_params=pltpu.CompilerParams(use_tc_tiling_on_sc=False),
    )
    def body(idx_hbm, table_hbm, out_hbm, idx0, idx1, rows0, rows1, in_sem,
             g_sem0, g_sem1, out_sem0, out_sem1):
        idxs = (idx0, idx1)
        rowss = (rows0, rows1)
        g_sems = (g_sem0, g_sem1)
        out_sems = (out_sem0, out_sem1)
        wid = lax.axis_index("s") * NC + lax.axis_index("c")
        base = wid * per_w

        def do_chunk(k, carry):
            cbase = base + k * chunk
            pltpu.sync_copy(idx_hbm.at[pl.ds(cbase, chunk)], idx0)

            def do_gather(j, c):
                sl = pl.ds(j * gath, gath)
                pltpu.async_copy(
                    table_hbm.at[idx0.at[sl]], rows0.at[sl], g_sem0
                ).wait()
                return c

            lax.fori_loop(0, chunk // gath, do_gather, 0)
            pltpu.sync_copy(rows0, out_hbm.at[pl.ds(cbase, chunk)])
            return carry

        lax.fori_loop(0, n_chunks, do_chunk, 0)

    return body


def kernel(xCat, table, offsets):
    B, F = xCat.shape
    V, D = table.shape
    N = B * F
    scratch, r_flat = _make_relayout(V, D, B, F)(
        table.T, xCat.T, offsets.astype(jnp.int32)
    )
    tail = V % 128
    if tail:
        tail_rows = table[V - tail :].reshape(tail * D // 128, 128)
        scratch = lax.dynamic_update_slice(
            scratch, tail_rows, ((V - tail) * D // 128, 0)
        )
    tbl = scratch.reshape(V, D)
    out = _make_gather(N, D, 8, 1664)(r_flat, tbl)
    return out.reshape(F, B, D).transpose(1, 0, 2)
